# baseline (speedup 1.0000x reference)
import functools
import math

import jax
import jax.numpy as jnp
from jax import lax
from jax.experimental import pallas as pl
from jax.experimental.pallas import tpu as pltpu
from jax.experimental.pallas import tpu_sc as plsc

N = 4096
L = 32
D = 256
H = 1024
NH = 8
DH = D // NH

BN = 256
G = 32


R = N * L
NW = 32
ROWS_W = R // NW
CSC = 256

_sc_mesh = plsc.VectorSubcoreMesh(core_axis_name="c", subcore_axis_name="s")


@functools.partial(
    pl.kernel, mesh=_sc_mesh,
    out_type=jax.ShapeDtypeStruct((R, D), jnp.float32),
    scratch_types=[
        pltpu.VMEM((CSC,), jnp.int32),
        pltpu.VMEM((CSC, D), jnp.float32),
        pltpu.SemaphoreType.DMA,
    ],
)
def _sc_shift(bank_hbm, idx_hbm, out_hbm, idx_v, rows_v, sem):
    wid = lax.axis_index("s") * 2 + lax.axis_index("c")
    base = wid * ROWS_W

    def chunk_body(ci, _):
        r0 = base + ci * CSC
        pltpu.sync_copy(idx_hbm.at[pl.ds(r0, CSC)], idx_v)
        pltpu.async_copy(bank_hbm.at[idx_v], rows_v, sem).wait()
        pltpu.sync_copy(rows_v, out_hbm.at[pl.ds(r0, CSC)])
        return 0

    lax.fori_loop(0, ROWS_W // CSC, chunk_body, 0)


def _ln(x, w, b):
    mu = jnp.mean(x, axis=-1, keepdims=True)
    var = jnp.mean((x - mu) ** 2, axis=-1, keepdims=True)
    return (x - mu) * jax.lax.rsqrt(var + 1e-5) * w + b


def _body(emb_ref, scores_ref, maskf_ref, maskg_ref, bank_ref,
          in_w_ref, in_b_ref, out_w_ref, out_b_ref,
          fc1_w_ref, fc1_b_ref, fc2_w_ref, fc2_b_ref,
          ln1_w_ref, ln1_b_ref, ln2_w_ref, ln2_b_ref,
          save_w_ref, save_b_ref,
          new_emb_ref, new_last_ref, new_maskf_ref):
    emb = emb_ref[...]
    maskf = maskf_ref[...]
    scores = scores_ref[...]

    valid = maskf[:, L - 1:L] < 0.5
    scale = 1.0 / math.sqrt(DH)

    q = jnp.dot(emb, in_w_ref[0:D, :].T, preferred_element_type=jnp.float32)
    q = q + in_b_ref[0, 0:D]

    qk_heads = [
        jnp.dot(q[:, h * DH:(h + 1) * DH],
                in_w_ref[D + h * DH:D + (h + 1) * DH, :],
                preferred_element_type=jnp.float32) * scale
        for h in range(NH)
    ]
    NG = BN // G
    row_g = jax.lax.broadcasted_iota(jnp.int32, (NH * G, G * L), 0) % G
    col_g = jax.lax.broadcasted_iota(jnp.int32, (NH * G, G * L), 1) // L
    off_diag = row_g != col_g

    saved = scores > 0.0
    save_rows = []
    mixed_gs = []
    for g in range(NG):
        qk_g = jnp.concatenate(
            [qk_heads[h][g * G:(g + 1) * G] for h in range(NH)], axis=0)
        bank_g3 = bank_ref[g * G:(g + 1) * G, :, :]
        bank_g = bank_g3.reshape(G * L, D)
        r = jax.lax.dot_general(
            qk_g, bank_g, (((1,), (1,)), ((), ())),
            preferred_element_type=jnp.float32)
        pad = maskg_ref[g:g + 1, :] > 0.5
        r = jnp.where(jnp.logical_or(off_diag, pad), -1e9, r)
        m = jnp.max(r, axis=-1, keepdims=True)
        p = jnp.exp(r - m)
        p = p / jnp.sum(p, axis=-1, keepdims=True)
        mixed_gs.append(jnp.dot(p, bank_g,
                                preferred_element_type=jnp.float32))
        save_rows.append(bank_g3[:, L - 1, :])

    ctx_heads = []
    for h in range(NH):
        mixed_h = jnp.concatenate(
            [mixed_gs[g][h * G:(h + 1) * G] for g in range(NG)], axis=0)
        wv_h = in_w_ref[2 * D + h * DH:2 * D + (h + 1) * DH, :]
        ctx_h = jnp.dot(mixed_h, wv_h.T,
                        preferred_element_type=jnp.float32)
        ctx_heads.append(ctx_h + in_b_ref[0, 2 * D + h * DH:2 * D + (h + 1) * DH])

    ctx = jnp.concatenate(ctx_heads, axis=1)
    attn_out = jnp.dot(ctx, out_w_ref[...].T,
                       preferred_element_type=jnp.float32) + out_b_ref[0]

    e = _ln(emb + attn_out, ln1_w_ref[0], ln1_b_ref[0])
    hmid = jnp.maximum(
        jnp.dot(e, fc1_w_ref[...].T, preferred_element_type=jnp.float32)
        + fc1_b_ref[0], 0.0)
    ffn = jnp.dot(hmid, fc2_w_ref[...].T,
                  preferred_element_type=jnp.float32) + fc2_b_ref[0]
    e2 = _ln(e + ffn, ln2_w_ref[0], ln2_b_ref[0])

    new_emb = jnp.where(valid, e2, emb)
    new_emb_ref[...] = new_emb

    save_embed = jnp.dot(new_emb, save_w_ref[...].T,
                         preferred_element_type=jnp.float32) + save_b_ref[0]

    last_old = jnp.concatenate(save_rows, axis=0)
    new_last_ref[...] = jnp.where(saved, save_embed, last_old)
    new_maskf_ref[:, :L - 1] = jnp.where(saved, maskf[:, 1:], maskf[:, :L - 1])
    new_maskf_ref[:, L - 1:L] = jnp.where(saved, 0.0, maskf[:, L - 1:L])


@functools.partial(jax.jit, static_argnames=("interpret",))
def _run(emb, scores2, maskf, maskg, bank, in_w, in_b2, out_w, out_b2,
         fc1_w, fc1_b2, fc2_w, fc2_b2, ln1_w2, ln1_b2, ln2_w2, ln2_b2,
         save_w, save_b2, *, interpret=False):
    grid = (N // BN,)
    row = lambda i: (i, 0)
    row3 = lambda i: (i, 0, 0)
    full = lambda i: (0, 0)
    out = pl.pallas_call(
        _body,
        grid=grid,
        in_specs=[
            pl.BlockSpec((BN, D), row),
            pl.BlockSpec((BN, 1), row),
            pl.BlockSpec((BN, L), row),
            pl.BlockSpec((BN // G, G * L), row),
            pl.BlockSpec((BN, L, D), row3),
            pl.BlockSpec((3 * D, D), full),
            pl.BlockSpec((1, 3 * D), full),
            pl.BlockSpec((D, D), full),
            pl.BlockSpec((1, D), full),
            pl.BlockSpec((H, D), full),
            pl.BlockSpec((1, H), full),
            pl.BlockSpec((D, H), full),
            pl.BlockSpec((1, D), full),
            pl.BlockSpec((1, D), full),
            pl.BlockSpec((1, D), full),
            pl.BlockSpec((1, D), full),
            pl.BlockSpec((1, D), full),
            pl.BlockSpec((D, D), full),
            pl.BlockSpec((1, D), full),
        ],
        out_specs=[
            pl.BlockSpec((BN, D), row),
            pl.BlockSpec((BN, D), row),
            pl.BlockSpec((BN, L), row),
        ],
        out_shape=[
            jax.ShapeDtypeStruct((N, D), jnp.float32),
            jax.ShapeDtypeStruct((N, D), jnp.float32),
            jax.ShapeDtypeStruct((N, L), jnp.float32),
        ],
        interpret=interpret,
    )(emb, scores2, maskf, maskg, bank, in_w, in_b2, out_w, out_b2,
      fc1_w, fc1_b2, fc2_w, fc2_b2, ln1_w2, ln1_b2, ln2_w2, ln2_b2,
      save_w, save_b2)
    return out


def kernel(output_embedding, scores, mem_padding_mask, mem_bank, save_period,
           save_proj_w, save_proj_b, in_proj_w, in_proj_b, out_proj_w,
           out_proj_b, fc1_w, fc1_b, fc2_w, fc2_b, ln1_w, ln1_b, ln2_w, ln2_b):
    del save_period
    maskf = mem_padding_mask.astype(jnp.float32)
    saved = scores > 0
    r = jnp.arange(R, dtype=jnp.int32)
    idx = r + (saved[r // L] & ((r % L) < L - 1)).astype(jnp.int32)
    nb1 = _sc_shift(mem_bank.reshape(R, D), idx)
    new_emb, new_last, new_maskf = _run(
        output_embedding, scores.reshape(N, 1), maskf,
        maskf.reshape(N // G, G * L), mem_bank,
        in_proj_w, in_proj_b.reshape(1, 3 * D), out_proj_w,
        out_proj_b.reshape(1, D), fc1_w, fc1_b.reshape(1, H), fc2_w,
        fc2_b.reshape(1, D), ln1_w.reshape(1, D), ln1_b.reshape(1, D),
        ln2_w.reshape(1, D), ln2_b.reshape(1, D), save_proj_w,
        save_proj_b.reshape(1, D))
    del new_last
    return new_emb, nb1.reshape(N, L, D), new_maskf > 0.5


# speedup vs baseline: 10.0611x; 10.0611x over previous
import functools
import math

import jax
import jax.numpy as jnp
from jax.experimental import pallas as pl

N = 4096
L = 32
D = 256
H = 1024
NH = 8
DH = D // NH

BN = 256
G = 32


def _ln(x, w, b):
    mu = jnp.mean(x, axis=-1, keepdims=True)
    var = jnp.mean((x - mu) ** 2, axis=-1, keepdims=True)
    return (x - mu) * jax.lax.rsqrt(var + 1e-5) * w + b


def _body(emb_ref, scores_ref, maskf_ref, maskg_ref, bank_ref,
          in_w_ref, in_b_ref, out_w_ref, out_b_ref,
          fc1_w_ref, fc1_b_ref, fc2_w_ref, fc2_b_ref,
          ln1_w_ref, ln1_b_ref, ln2_w_ref, ln2_b_ref,
          save_w_ref, save_b_ref,
          new_emb_ref, new_bank_ref, new_maskf_ref):
    emb = emb_ref[...]
    maskf = maskf_ref[...]
    scores = scores_ref[...]

    valid = maskf[:, L - 1:L] < 0.5
    scale = 1.0 / math.sqrt(DH)

    q = jnp.dot(emb, in_w_ref[0:D, :].T, preferred_element_type=jnp.float32)
    q = q + in_b_ref[0, 0:D]

    qk_heads = [
        jnp.dot(q[:, h * DH:(h + 1) * DH],
                in_w_ref[D + h * DH:D + (h + 1) * DH, :],
                preferred_element_type=jnp.float32) * scale
        for h in range(NH)
    ]
    NG = BN // G
    row_g = jax.lax.broadcasted_iota(jnp.int32, (NH * G, G * L), 0) % G
    col_g = jax.lax.broadcasted_iota(jnp.int32, (NH * G, G * L), 1) // L
    off_diag = row_g != col_g

    saved = scores > 0.0
    save_rows = []
    mixed_gs = []
    for g in range(NG):
        qk_g = jnp.concatenate(
            [qk_heads[h][g * G:(g + 1) * G] for h in range(NH)], axis=0)
        bank_g3 = bank_ref[g * G:(g + 1) * G, :, :]
        bank_g = bank_g3.reshape(G * L, D)
        r = jax.lax.dot_general(
            qk_g, bank_g, (((1,), (1,)), ((), ())),
            preferred_element_type=jnp.float32)
        pad = maskg_ref[g:g + 1, :] > 0.5
        r = jnp.where(jnp.logical_or(off_diag, pad), -1e9, r)
        m = jnp.max(r, axis=-1, keepdims=True)
        p = jnp.exp(r - m)
        p = p / jnp.sum(p, axis=-1, keepdims=True)
        mixed_gs.append(jnp.dot(p, bank_g,
                                preferred_element_type=jnp.float32))
        saved3 = saved[g * G:(g + 1) * G, :, None]
        new_bank_ref[g * G:(g + 1) * G, :L - 1, :] = jnp.where(
            saved3, bank_g3[:, 1:, :], bank_g3[:, :L - 1, :])
        save_rows.append(bank_g3[:, L - 1, :])

    ctx_heads = []
    for h in range(NH):
        mixed_h = jnp.concatenate(
            [mixed_gs[g][h * G:(h + 1) * G] for g in range(NG)], axis=0)
        wv_h = in_w_ref[2 * D + h * DH:2 * D + (h + 1) * DH, :]
        ctx_h = jnp.dot(mixed_h, wv_h.T,
                        preferred_element_type=jnp.float32)
        ctx_heads.append(ctx_h + in_b_ref[0, 2 * D + h * DH:2 * D + (h + 1) * DH])

    ctx = jnp.concatenate(ctx_heads, axis=1)
    attn_out = jnp.dot(ctx, out_w_ref[...].T,
                       preferred_element_type=jnp.float32) + out_b_ref[0]

    e = _ln(emb + attn_out, ln1_w_ref[0], ln1_b_ref[0])
    hmid = jnp.maximum(
        jnp.dot(e, fc1_w_ref[...].T, preferred_element_type=jnp.float32)
        + fc1_b_ref[0], 0.0)
    ffn = jnp.dot(hmid, fc2_w_ref[...].T,
                  preferred_element_type=jnp.float32) + fc2_b_ref[0]
    e2 = _ln(e + ffn, ln2_w_ref[0], ln2_b_ref[0])

    new_emb = jnp.where(valid, e2, emb)
    new_emb_ref[...] = new_emb

    save_embed = jnp.dot(new_emb, save_w_ref[...].T,
                         preferred_element_type=jnp.float32) + save_b_ref[0]

    last_old = jnp.concatenate(save_rows, axis=0)
    new_last = jnp.where(saved, save_embed, last_old)
    new_bank_ref[:, L - 1:L, :] = new_last[:, None, :]
    new_maskf_ref[:, :L - 1] = jnp.where(saved, maskf[:, 1:], maskf[:, :L - 1])
    new_maskf_ref[:, L - 1:L] = jnp.where(saved, 0.0, maskf[:, L - 1:L])


@jax.jit
def _run(emb, scores2, maskf, maskg, bank, in_w, in_b2, out_w, out_b2,
         fc1_w, fc1_b2, fc2_w, fc2_b2, ln1_w2, ln1_b2, ln2_w2, ln2_b2,
         save_w, save_b2):
    grid = (N // BN,)
    row = lambda i: (i, 0)
    row3 = lambda i: (i, 0, 0)
    full = lambda i: (0, 0)
    out = pl.pallas_call(
        _body,
        grid=grid,
        in_specs=[
            pl.BlockSpec((BN, D), row),
            pl.BlockSpec((BN, 1), row),
            pl.BlockSpec((BN, L), row),
            pl.BlockSpec((BN // G, G * L), row),
            pl.BlockSpec((BN, L, D), row3),
            pl.BlockSpec((3 * D, D), full),
            pl.BlockSpec((1, 3 * D), full),
            pl.BlockSpec((D, D), full),
            pl.BlockSpec((1, D), full),
            pl.BlockSpec((H, D), full),
            pl.BlockSpec((1, H), full),
            pl.BlockSpec((D, H), full),
            pl.BlockSpec((1, D), full),
            pl.BlockSpec((1, D), full),
            pl.BlockSpec((1, D), full),
            pl.BlockSpec((1, D), full),
            pl.BlockSpec((1, D), full),
            pl.BlockSpec((D, D), full),
            pl.BlockSpec((1, D), full),
        ],
        out_specs=[
            pl.BlockSpec((BN, D), row),
            pl.BlockSpec((BN, L, D), row3),
            pl.BlockSpec((BN, L), row),
        ],
        out_shape=[
            jax.ShapeDtypeStruct((N, D), jnp.float32),
            jax.ShapeDtypeStruct((N, L, D), jnp.float32),
            jax.ShapeDtypeStruct((N, L), jnp.float32),
        ],
    )(emb, scores2, maskf, maskg, bank, in_w, in_b2, out_w, out_b2,
      fc1_w, fc1_b2, fc2_w, fc2_b2, ln1_w2, ln1_b2, ln2_w2, ln2_b2,
      save_w, save_b2)
    return out


def kernel(output_embedding, scores, mem_padding_mask, mem_bank, save_period,
           save_proj_w, save_proj_b, in_proj_w, in_proj_b, out_proj_w,
           out_proj_b, fc1_w, fc1_b, fc2_w, fc2_b, ln1_w, ln1_b, ln2_w, ln2_b):
    del save_period
    maskf = mem_padding_mask.astype(jnp.float32)
    new_emb, new_bank, new_maskf = _run(
        output_embedding, scores.reshape(N, 1), maskf,
        maskf.reshape(N // G, G * L), mem_bank,
        in_proj_w, in_proj_b.reshape(1, 3 * D), out_proj_w,
        out_proj_b.reshape(1, D), fc1_w, fc1_b.reshape(1, H), fc2_w,
        fc2_b.reshape(1, D), ln1_w.reshape(1, D), ln1_b.reshape(1, D),
        ln2_w.reshape(1, D), ln2_b.reshape(1, D), save_proj_w,
        save_proj_b.reshape(1, D))
    return new_emb, new_bank, new_maskf > 0.5
